# baseline (device time: 96821 ns/iter reference)
import jax
import jax.numpy as jnp
from jax import lax
from jax.experimental import pallas as pl
from jax.experimental.pallas import tpu as pltpu

N_DEV = 4


def kernel(x, W1, W2):
    m, _ = x.shape
    n = W2.shape[1]

    def body(x_ref, w1_ref, w2_ref, out_ref, comm_ref, send_sems, recv_sems):
        my = lax.axis_index("i")
        left = lax.rem(my + N_DEV - 1, N_DEV)
        right = lax.rem(my + 1, N_DEV)

        barrier_sem = pltpu.get_barrier_semaphore()
        for nbr in (left, right):
            pl.semaphore_signal(
                barrier_sem, inc=1,
                device_id=(nbr,), device_id_type=pl.DeviceIdType.MESH,
            )
        pl.semaphore_wait(barrier_sem, 2)

        h = jnp.maximum(
            jnp.dot(x_ref[...], w1_ref[...], preferred_element_type=jnp.float32),
            0.0,
        )
        partial = jnp.dot(h, w2_ref[...], preferred_element_type=jnp.float32)
        comm_ref[0] = partial

        acc = partial
        for hop in range(N_DEV - 1):
            rdma = pltpu.make_async_remote_copy(
                src_ref=comm_ref.at[hop],
                dst_ref=comm_ref.at[hop + 1],
                send_sem=send_sems.at[hop],
                recv_sem=recv_sems.at[hop + 1],
                device_id=(right,),
                device_id_type=pl.DeviceIdType.MESH,
            )
            rdma.start()
            rdma.wait()
            acc = acc + comm_ref[hop + 1]
        out_ref[...] = acc

    return pl.pallas_call(
        body,
        out_shape=jax.ShapeDtypeStruct((m, n), jnp.float32),
        in_specs=[pl.BlockSpec(memory_space=pltpu.VMEM)] * 3,
        out_specs=pl.BlockSpec(memory_space=pltpu.VMEM),
        scratch_shapes=[
            pltpu.VMEM((N_DEV, m, n), jnp.float32),
            pltpu.SemaphoreType.DMA((N_DEV,)),
            pltpu.SemaphoreType.DMA((N_DEV,)),
        ],
        compiler_params=pltpu.CompilerParams(collective_id=0),
    )(x, W1, W2)


# device time: 40814 ns/iter; 2.3722x vs baseline; 2.3722x over previous
import jax
import jax.numpy as jnp
from jax import lax
from jax.experimental import pallas as pl
from jax.experimental.pallas import tpu as pltpu

N_DEV = 4
M = 768
H2 = M // 2
Q = M // 4
E = M // 8


def kernel(x, W1, W2):
    m, _ = x.shape
    n = W2.shape[1]

    def body(x_ref, w1_ref, w2_ref, out_ref,
             recv_a1, recv_b1, recv_a2, recv_b2, send_sems, recv_sems):
        p = lax.axis_index("i")
        partner1 = p ^ 1
        partner2 = 3 - p
        b = (p ^ (p >> 1)) & 1
        c = (p >> 1) & 1
        d = p & 1

        barrier_sem = pltpu.get_barrier_semaphore()
        for nbr in (partner1, partner2):
            pl.semaphore_signal(
                barrier_sem, inc=1,
                device_id=(nbr,), device_id_type=pl.DeviceIdType.MESH,
            )
        pl.semaphore_wait(barrier_sem, 2)

        h = jnp.maximum(
            jnp.dot(x_ref[...], w1_ref[...], preferred_element_type=jnp.float32),
            0.0,
        )
        out_ref[...] = jnp.dot(h, w2_ref[...], preferred_element_type=jnp.float32)

        a1 = b * Q
        b1 = H2 + c * Q

        def xchg(idx, src_slc, dst_ref, dst_slc, dev):
            return pltpu.make_async_remote_copy(
                src_ref=out_ref.at[src_slc],
                dst_ref=dst_ref.at[dst_slc] if dst_slc is not None else dst_ref,
                send_sem=send_sems.at[idx],
                recv_sem=recv_sems.at[idx],
                device_id=(dev,),
                device_id_type=pl.DeviceIdType.MESH,
            )

        e0 = xchg(0, pl.ds((1 - b) * Q, Q), recv_a1, None, partner1)
        e1 = xchg(1, pl.ds(H2 + (1 - c) * Q, Q), recv_b1, None, partner2)
        e0.start()
        e1.start()
        e0.wait()
        e1.wait()
        out_ref[pl.ds(a1, Q)] += recv_a1[...]
        out_ref[pl.ds(b1, Q)] += recv_b1[...]

        e2 = xchg(2, pl.ds(a1 + (1 - c) * E, E), recv_a2, None, partner2)
        e3 = xchg(3, pl.ds(b1 + (1 - d) * E, E), recv_b2, None, partner1)
        e2.start()
        e3.start()
        e2.wait()
        e3.wait()
        out_ref[pl.ds(a1 + c * E, E)] += recv_a2[...]
        out_ref[pl.ds(b1 + d * E, E)] += recv_b2[...]

        e4 = xchg(4, pl.ds(a1 + c * E, E), out_ref, pl.ds(a1 + c * E, E), partner2)
        e5 = xchg(5, pl.ds(b1 + d * E, E), out_ref, pl.ds(b1 + d * E, E), partner1)
        e4.start()
        e5.start()
        e4.wait()
        e5.wait()

        e6 = xchg(6, pl.ds(a1, Q), out_ref, pl.ds(a1, Q), partner1)
        e7 = xchg(7, pl.ds(b1, Q), out_ref, pl.ds(b1, Q), partner2)
        e6.start()
        e7.start()
        e6.wait()
        e7.wait()

    return pl.pallas_call(
        body,
        out_shape=jax.ShapeDtypeStruct((m, n), jnp.float32),
        in_specs=[pl.BlockSpec(memory_space=pltpu.VMEM)] * 3,
        out_specs=pl.BlockSpec(memory_space=pltpu.VMEM),
        scratch_shapes=[
            pltpu.VMEM((Q, n), jnp.float32),
            pltpu.VMEM((Q, n), jnp.float32),
            pltpu.VMEM((E, n), jnp.float32),
            pltpu.VMEM((E, n), jnp.float32),
            pltpu.SemaphoreType.DMA((8,)),
            pltpu.SemaphoreType.DMA((8,)),
        ],
        compiler_params=pltpu.CompilerParams(collective_id=0),
    )(x, W1, W2)


# device time: 39108 ns/iter; 2.4757x vs baseline; 1.0436x over previous
import jax
import jax.numpy as jnp
from jax import lax
from jax.experimental import pallas as pl
from jax.experimental.pallas import tpu as pltpu

N_DEV = 4
M = 768
H2 = M // 2
Q = M // 4
E = M // 8


def kernel(x, W1, W2):
    m, _ = x.shape
    n = W2.shape[1]

    def body(x_ref, w1_ref, w2_ref, out_ref,
             recv_a1, recv_b1, recv_a2, recv_b2, send_sems, recv_sems):
        p = lax.axis_index("i")
        partner1 = p ^ 1
        partner2 = 3 - p
        b = (p ^ (p >> 1)) & 1
        c = (p >> 1) & 1
        d = p & 1

        barrier_sem = pltpu.get_barrier_semaphore()
        for nbr in (partner1, partner2):
            pl.semaphore_signal(
                barrier_sem, inc=1,
                device_id=(nbr,), device_id_type=pl.DeviceIdType.MESH,
            )
        pl.semaphore_wait(barrier_sem, 2)

        a1 = b * Q
        b1 = H2 + c * Q

        def mlp_rows(start):
            xa = x_ref[pl.ds(start, Q)]
            hh = jnp.maximum(
                jnp.dot(xa, w1_ref[...], preferred_element_type=jnp.float32),
                0.0,
            )
            out_ref[pl.ds(start, Q)] = jnp.dot(
                hh, w2_ref[...], preferred_element_type=jnp.float32
            )

        def xchg(idx, src_slc, dst_ref, dst_slc, dev):
            return pltpu.make_async_remote_copy(
                src_ref=out_ref.at[src_slc],
                dst_ref=dst_ref.at[dst_slc] if dst_slc is not None else dst_ref,
                send_sem=send_sems.at[idx],
                recv_sem=recv_sems.at[idx],
                device_id=(dev,),
                device_id_type=pl.DeviceIdType.MESH,
            )

        mlp_rows((1 - b) * Q)
        e0 = xchg(0, pl.ds((1 - b) * Q, Q), recv_a1, None, partner1)
        e0.start()
        mlp_rows(H2 + (1 - c) * Q)
        e1 = xchg(1, pl.ds(H2 + (1 - c) * Q, Q), recv_b1, None, partner2)
        e1.start()
        mlp_rows(a1)
        mlp_rows(b1)

        e2 = xchg(2, pl.ds(a1 + (1 - c) * E, E), recv_a2, None, partner2)
        e3 = xchg(3, pl.ds(b1 + (1 - d) * E, E), recv_b2, None, partner1)
        e4 = xchg(4, pl.ds(a1 + c * E, E), out_ref, pl.ds(a1 + c * E, E), partner2)
        e5 = xchg(5, pl.ds(b1 + d * E, E), out_ref, pl.ds(b1 + d * E, E), partner1)
        e6 = xchg(6, pl.ds(a1, Q), out_ref, pl.ds(a1, Q), partner1)
        e7 = xchg(7, pl.ds(b1, Q), out_ref, pl.ds(b1, Q), partner2)

        e0.wait()
        out_ref[pl.ds(a1, Q)] += recv_a1[...]
        e2.start()
        e1.wait()
        out_ref[pl.ds(b1, Q)] += recv_b1[...]
        e3.start()
        e2.wait()
        out_ref[pl.ds(a1 + c * E, E)] += recv_a2[...]
        e4.start()
        e3.wait()
        out_ref[pl.ds(b1 + d * E, E)] += recv_b2[...]
        e5.start()
        e4.wait()
        e6.start()
        e5.wait()
        e7.start()
        e6.wait()
        e7.wait()

    return pl.pallas_call(
        body,
        out_shape=jax.ShapeDtypeStruct((m, n), jnp.float32),
        in_specs=[pl.BlockSpec(memory_space=pltpu.VMEM)] * 3,
        out_specs=pl.BlockSpec(memory_space=pltpu.VMEM),
        scratch_shapes=[
            pltpu.VMEM((Q, n), jnp.float32),
            pltpu.VMEM((Q, n), jnp.float32),
            pltpu.VMEM((E, n), jnp.float32),
            pltpu.VMEM((E, n), jnp.float32),
            pltpu.SemaphoreType.DMA((8,)),
            pltpu.SemaphoreType.DMA((8,)),
        ],
        compiler_params=pltpu.CompilerParams(collective_id=0),
    )(x, W1, W2)


# device time: 30906 ns/iter; 3.1328x vs baseline; 1.2654x over previous
import jax
import jax.numpy as jnp
from jax import lax
from jax.experimental import pallas as pl
from jax.experimental.pallas import tpu as pltpu

N_DEV = 4
M = 768
H2 = M // 2
Q = M // 4
E = M // 8

F32 = jnp.float32
BF16 = jnp.bfloat16


def kernel(x, W1, W2):
    m, _ = x.shape
    k2, n = W2.shape

    def body(x_ref, w1_ref, w2_ref, out_ref, h_ref,
             sa1, sb1, ra1, rb1, sa2, sb2, ra2, rb2, ga, gb, gar, gbr,
             send_sems, recv_sems):
        p = lax.axis_index("i")
        partner1 = p ^ 1
        partner2 = 3 - p
        b = (p ^ (p >> 1)) & 1
        c = (p >> 1) & 1
        d = p & 1
        a1 = b * Q
        b1 = H2 + c * Q

        barrier_sem = pltpu.get_barrier_semaphore()
        for nbr in (partner1, partner2):
            pl.semaphore_signal(
                barrier_sem, inc=1,
                device_id=(nbr,), device_id_type=pl.DeviceIdType.MESH,
            )
        pl.semaphore_wait(barrier_sem, 2)

        def xchg(idx, src_ref, src_slc, dst_ref, dst_slc, dev):
            return pltpu.make_async_remote_copy(
                src_ref=src_ref.at[src_slc] if src_slc is not None else src_ref,
                dst_ref=dst_ref.at[dst_slc] if dst_slc is not None else dst_ref,
                send_sem=send_sems.at[idx],
                recv_sem=recv_sems.at[idx],
                device_id=(dev,),
                device_id_type=pl.DeviceIdType.MESH,
            )

        h_ref[...] = jnp.maximum(
            jnp.dot(x_ref[...], w1_ref[...], preferred_element_type=F32), 0.0
        )

        def mm2(start):
            return jnp.dot(
                h_ref[pl.ds(start, Q)], w2_ref[...], preferred_element_type=F32
            )

        sa1[...] = mm2((1 - b) * Q).astype(BF16)
        e0 = xchg(0, sa1, None, ra1, None, partner1)
        e0.start()
        sb1[...] = mm2(H2 + (1 - c) * Q).astype(BF16)
        e1 = xchg(1, sb1, None, rb1, None, partner2)
        e1.start()
        out_ref[pl.ds(a1, Q)] = mm2(a1)
        out_ref[pl.ds(b1, Q)] = mm2(b1)

        e2 = xchg(2, sa2, None, ra2, None, partner2)
        e3 = xchg(3, sb2, None, rb2, None, partner1)
        e4 = xchg(4, ga, pl.ds(c * E, E), ga, pl.ds(c * E, E), partner2)
        e5 = xchg(5, gb, pl.ds(d * E, E), gb, pl.ds(d * E, E), partner1)
        e6 = xchg(6, ga, None, gar, None, partner1)
        e7 = xchg(7, gb, None, gbr, None, partner2)

        e0.wait()
        out_ref[pl.ds(a1, Q)] += ra1[...].astype(F32)
        sa2[...] = out_ref[pl.ds(a1 + (1 - c) * E, E)].astype(BF16)
        e2.start()
        e1.wait()
        out_ref[pl.ds(b1, Q)] += rb1[...].astype(F32)
        sb2[...] = out_ref[pl.ds(b1 + (1 - d) * E, E)].astype(BF16)
        e3.start()
        e2.wait()
        out_ref[pl.ds(a1 + c * E, E)] += ra2[...].astype(F32)
        ga[pl.ds(c * E, E)] = out_ref[pl.ds(a1 + c * E, E)].astype(BF16)
        e4.start()
        e3.wait()
        out_ref[pl.ds(b1 + d * E, E)] += rb2[...].astype(F32)
        gb[pl.ds(d * E, E)] = out_ref[pl.ds(b1 + d * E, E)].astype(BF16)
        e5.start()
        e4.wait()
        e6.start()
        e5.wait()
        e7.start()
        e6.wait()
        out_ref[pl.ds(a1 + (1 - c) * E, E)] = ga[pl.ds((1 - c) * E, E)].astype(F32)
        out_ref[pl.ds((1 - b) * Q, Q)] = gar[...].astype(F32)
        e7.wait()
        out_ref[pl.ds(b1 + (1 - d) * E, E)] = gb[pl.ds((1 - d) * E, E)].astype(F32)
        out_ref[pl.ds(H2 + (1 - c) * Q, Q)] = gbr[...].astype(F32)

    return pl.pallas_call(
        body,
        out_shape=jax.ShapeDtypeStruct((m, n), F32),
        in_specs=[pl.BlockSpec(memory_space=pltpu.VMEM)] * 3,
        out_specs=pl.BlockSpec(memory_space=pltpu.VMEM),
        scratch_shapes=[
            pltpu.VMEM((m, k2), F32),
            pltpu.VMEM((Q, n), BF16),
            pltpu.VMEM((Q, n), BF16),
            pltpu.VMEM((Q, n), BF16),
            pltpu.VMEM((Q, n), BF16),
            pltpu.VMEM((E, n), BF16),
            pltpu.VMEM((E, n), BF16),
            pltpu.VMEM((E, n), BF16),
            pltpu.VMEM((E, n), BF16),
            pltpu.VMEM((Q, n), BF16),
            pltpu.VMEM((Q, n), BF16),
            pltpu.VMEM((Q, n), BF16),
            pltpu.VMEM((Q, n), BF16),
            pltpu.SemaphoreType.DMA((8,)),
            pltpu.SemaphoreType.DMA((8,)),
        ],
        compiler_params=pltpu.CompilerParams(collective_id=0),
    )(x, W1, W2)
